# Initial kernel scaffold; baseline (speedup 1.0000x reference)
import jax
import jax.numpy as jnp
from jax.experimental import pallas as pl


def kernel(x, edge_index, edge_attr, W1, b1, W2, b2, Wfc, bfc):
    raise NotImplementedError("write your pallas kernel here")


# speedup vs baseline: 8.2989x; 8.2989x over previous
import functools

import jax
import jax.numpy as jnp
from jax import lax
from jax.experimental import pallas as pl
from jax.experimental.pallas import tpu as pltpu
from jax.experimental.pallas import tpu_sc as plsc

N = 10000
NP = 10240
D = 128
NCLS = 64

NC = 2
NS = 16
NW = NC * NS
CH = 128

RPT = NP // NS

R = 1024
G = NP // R

_MESH = dict(core_axis_name="c", subcore_axis_name="s")


def _deg_body(ew, nch, dst_hbm, w_hbm, degp_hbm, acc_v, dst_v, w_v):
    cid = lax.axis_index("c")
    sid = lax.axis_index("s")
    wid = sid * NC + cid

    def zero(i, c):
        acc_v[pl.ds(i * 16, 16)] = jnp.zeros((16,), jnp.float32)
        return c

    lax.fori_loop(0, NP // 16, zero, 0)

    base = wid * ew

    def chunk(ci, c):
        off = base + ci * CH
        pltpu.sync_copy(dst_hbm.at[pl.ds(off, CH)], dst_v)
        pltpu.sync_copy(w_hbm.at[pl.ds(off, CH)], w_v)

        def grp(g, cc):
            idxv = dst_v[pl.ds(g * 16, 16)]
            wv = w_v[pl.ds(g * 16, 16)]
            plsc.addupdate_scatter(acc_v, [idxv], wv)
            return cc

        lax.fori_loop(0, CH // 16, grp, 0)
        return c

    lax.fori_loop(0, nch, chunk, 0)
    pltpu.sync_copy(acc_v, degp_hbm.at[wid])


def _make_sc_deg(ep):
    ew = ep // NW
    nch = ew // CH
    return pl.kernel(
        functools.partial(_deg_body, ew, nch),
        out_type=jax.ShapeDtypeStruct((NW, NP), jnp.float32),
        mesh=plsc.VectorSubcoreMesh(**_MESH),
        compiler_params=pltpu.CompilerParams(needs_layout_passes=False),
        scratch_types=[
            pltpu.VMEM((NP,), jnp.float32),
            pltpu.VMEM((CH,), jnp.int32),
            pltpu.VMEM((CH,), jnp.float32),
        ],
    )


def _edge_body(ew, nch, hp_hbm, src_hbm, dst_hbm, w_hbm, accp_hbm,
               acc_s, rows_v, src_v, dst_v, w_v, sem):
    cid = lax.axis_index("c")
    sid = lax.axis_index("s")
    wid = sid * NC + cid

    def zrow(r, c):
        for k in range(D // 16):
            rows_v[r, pl.ds(k * 16, 16)] = jnp.zeros((16,), jnp.float32)
        return c

    lax.fori_loop(0, CH, zrow, 0)
    for z in range(RPT // CH):
        pltpu.sync_copy(rows_v, acc_s.at[pl.ds(sid * RPT + z * CH, CH)])
    plsc.subcore_barrier()

    base = wid * ew

    def chunk(ci, c):
        off = base + ci * CH
        pltpu.sync_copy(src_hbm.at[pl.ds(off, CH)], src_v)
        pltpu.sync_copy(dst_hbm.at[pl.ds(off, CH)], dst_v)
        pltpu.sync_copy(w_hbm.at[pl.ds(off, CH)], w_v)
        pltpu.async_copy(hp_hbm.at[src_v], rows_v, sem).wait()

        def scale(r, cc):
            ws = plsc.load_gather(w_v, [jnp.full((16,), r, jnp.int32)])
            for k in range(D // 16):
                sl = pl.ds(k * 16, 16)
                rows_v[r, sl] = rows_v[r, sl] * ws
            return cc

        lax.fori_loop(0, CH, scale, 0)
        pltpu.sync_copy(rows_v, acc_s.at[dst_v], add=True)
        return c

    lax.fori_loop(0, nch, chunk, 0)
    plsc.subcore_barrier()
    pltpu.sync_copy(acc_s.at[pl.ds(sid * RPT, RPT)],
                    accp_hbm.at[cid, pl.ds(sid * RPT, RPT)])


def _make_sc_edges(ep):
    ew = ep // NW
    nch = ew // CH
    return pl.kernel(
        functools.partial(_edge_body, ew, nch),
        out_type=jax.ShapeDtypeStruct((NC, NP, D), jnp.float32),
        mesh=plsc.VectorSubcoreMesh(**_MESH),
        compiler_params=pltpu.CompilerParams(needs_layout_passes=False),
        scratch_types=[
            pltpu.VMEM_SHARED((NP, D), jnp.float32),
            pltpu.VMEM((CH, D), jnp.float32),
            pltpu.VMEM((CH,), jnp.int32),
            pltpu.VMEM((CH,), jnp.int32),
            pltpu.VMEM((CH,), jnp.float32),
            pltpu.SemaphoreType.DMA,
        ],
    )


def _tc_prep_body(degp_ref, x_ref, w_ref, hp_ref, disb_ref):
    deg = jnp.sum(degp_ref[...], axis=0) + 1.0
    dis = jnp.where(deg > 0, lax.rsqrt(jnp.maximum(deg, 1e-12)), 0.0)
    h = jnp.dot(x_ref[...], w_ref[...], preferred_element_type=jnp.float32,
                precision=lax.Precision.HIGHEST)
    d2 = jnp.broadcast_to(dis[:, None], (R, D))
    hp_ref[...] = h * d2
    disb_ref[...] = d2


def _tc_prep(degp, x, w1):
    return pl.pallas_call(
        _tc_prep_body,
        grid=(G,),
        in_specs=[
            pl.BlockSpec((NW, R), lambda i: (0, i)),
            pl.BlockSpec((R, D), lambda i: (i, 0)),
            pl.BlockSpec((D, D), lambda i: (0, 0)),
        ],
        out_specs=[
            pl.BlockSpec((R, D), lambda i: (i, 0)),
            pl.BlockSpec((R, D), lambda i: (i, 0)),
        ],
        out_shape=[
            jax.ShapeDtypeStruct((NP, D), jnp.float32),
            jax.ShapeDtypeStruct((NP, D), jnp.float32),
        ],
    )(degp, x, w1)


def _tc_mid_body(accp_ref, hp_ref, disb_ref, b_ref, w_ref, out_ref):
    acc = accp_ref[0] + accp_ref[1]
    z = jnp.maximum(disb_ref[...] * (acc + hp_ref[...]) + b_ref[...], 0.0)
    out_ref[...] = disb_ref[...] * jnp.dot(
        z, w_ref[...], preferred_element_type=jnp.float32,
        precision=lax.Precision.HIGHEST)


def _tc_mid(accp, hp, disb, b, w2):
    return pl.pallas_call(
        _tc_mid_body,
        grid=(G,),
        in_specs=[
            pl.BlockSpec((NC, R, D), lambda i: (0, i, 0)),
            pl.BlockSpec((R, D), lambda i: (i, 0)),
            pl.BlockSpec((R, D), lambda i: (i, 0)),
            pl.BlockSpec((1, D), lambda i: (0, 0)),
            pl.BlockSpec((D, D), lambda i: (0, 0)),
        ],
        out_specs=pl.BlockSpec((R, D), lambda i: (i, 0)),
        out_shape=jax.ShapeDtypeStruct((NP, D), jnp.float32),
    )(accp, hp, disb, b.reshape(1, D), w2)


def _tc_final_body(accp_ref, hp_ref, disb_ref, b_ref, w_ref, bfc_ref, out_ref):
    acc = accp_ref[0] + accp_ref[1]
    z = jnp.maximum(disb_ref[...] * (acc + hp_ref[...]) + b_ref[...], 0.0)
    out_ref[...] = jnp.dot(z, w_ref[...], preferred_element_type=jnp.float32,
                           precision=lax.Precision.HIGHEST) + bfc_ref[...]


def _tc_final(accp, hp, disb, b, wfc, bfc):
    return pl.pallas_call(
        _tc_final_body,
        grid=(G,),
        in_specs=[
            pl.BlockSpec((NC, R, D), lambda i: (0, i, 0)),
            pl.BlockSpec((R, D), lambda i: (i, 0)),
            pl.BlockSpec((R, D), lambda i: (i, 0)),
            pl.BlockSpec((1, D), lambda i: (0, 0)),
            pl.BlockSpec((D, NCLS), lambda i: (0, 0)),
            pl.BlockSpec((1, NCLS), lambda i: (0, 0)),
        ],
        out_specs=pl.BlockSpec((R, NCLS), lambda i: (i, 0)),
        out_shape=jax.ShapeDtypeStruct((NP, NCLS), jnp.float32),
    )(accp, hp, disb, b.reshape(1, D), wfc, bfc.reshape(1, NCLS))


def kernel(x, edge_index, edge_attr, W1, b1, W2, b2, Wfc, bfc):
    e = edge_index.shape[1]
    ep = -(-e // (NW * CH)) * (NW * CH)
    pad = ep - e
    src = jnp.concatenate([edge_index[0], jnp.zeros((pad,), edge_index.dtype)])
    dst = jnp.concatenate([edge_index[1], jnp.zeros((pad,), edge_index.dtype)])
    w = jnp.concatenate([edge_attr, jnp.zeros((pad,), edge_attr.dtype)])
    xp = jnp.pad(x, ((0, NP - x.shape[0]), (0, 0)))

    sc_deg = _make_sc_deg(ep)
    sc_edges = _make_sc_edges(ep)

    degp = sc_deg(dst, w)
    hp1, disb = _tc_prep(degp, xp, W1)
    accp1 = sc_edges(hp1, src, dst, w)
    hp2 = _tc_mid(accp1, hp1, disb, b1, W2)
    accp2 = sc_edges(hp2, src, dst, w)
    return _tc_final(accp2, hp2, disb, b2, Wfc, bfc)[:N]
